# Initial kernel scaffold; baseline (speedup 1.0000x reference)
import jax
import jax.numpy as jnp
from jax.experimental import pallas as pl


def kernel(x, edge_index, edge_type, basis, comp, root, bias, gamma, beta):
    raise NotImplementedError("write your pallas kernel here")


# speedup vs baseline: 8.2830x; 8.2830x over previous
import functools

import jax
import jax.numpy as jnp
from jax import lax
from jax.experimental import pallas as pl
from jax.experimental.pallas import tpu as pltpu
from jax.experimental.pallas import tpu_sc as plsc

N_NODES = 10000
N_EDGES = 320000
IN_CH = 128
HID_CH = 256
NUM_REL = 24
REL_PAD = 32
SEGS = N_NODES * REL_PAD
N_PAD = 10240

NC = 2
NS = 16
LANES = 16
CHUNK = 80

EDGES_PER_TILE_CNT = N_EDGES // NS
EDGES_PER_TILE = N_EDGES // (NC * NS)
CNT_CHUNKS = EDGES_PER_TILE_CNT // CHUNK
MAIN_CHUNKS = EDGES_PER_TILE // CHUNK
TBL_PER_TILE = SEGS // NS
TBUF = 4000
ROWS_PER_TILE = N_PAD // NS


def _sc_body(src_hbm, dst_hbm, et_hbm, comp_hbm, x_hbm,
             out_hbm, z_sp, tbl_sp, rows_v, srcv, dstv, etv, segv, coefv,
             onesv, compv, tbuf):
    c = lax.axis_index("c")
    s = lax.axis_index("s")

    zero = jnp.zeros((LANES,), jnp.float32)

    def zero_rows(e, _):
        for j in range(IN_CH // LANES):
            rows_v[e, pl.ds(j * LANES, LANES)] = zero
        return 0

    lax.fori_loop(0, CHUNK, zero_rows, 0)

    def zero_tbuf(j, _):
        tbuf[pl.ds(j * LANES, LANES)] = zero
        return 0

    lax.fori_loop(0, TBUF // LANES, zero_tbuf, 0)
    for q in range(TBL_PER_TILE // TBUF):
        pltpu.sync_copy(tbuf, tbl_sp.at[pl.ds(s * TBL_PER_TILE + q * TBUF,
                                              TBUF)])
    for k in range(ROWS_PER_TILE // CHUNK):
        pltpu.sync_copy(
            rows_v, z_sp.at[pl.ds(s * ROWS_PER_TILE + k * CHUNK, CHUNK)])
    pltpu.sync_copy(comp_hbm, compv)
    ones = jnp.ones((LANES,), jnp.float32)
    for j in range(CHUNK // LANES):
        onesv[pl.ds(j * LANES, LANES)] = ones
    plsc.subcore_barrier()

    def count_body(i, _):
        base = s * EDGES_PER_TILE_CNT + i * CHUNK
        pltpu.sync_copy(dst_hbm.at[pl.ds(base, CHUNK)], dstv)
        pltpu.sync_copy(et_hbm.at[pl.ds(base, CHUNK)], etv)
        for j in range(CHUNK // LANES):
            sl = pl.ds(j * LANES, LANES)
            segv[sl] = lax.shift_left(dstv[sl], 5) + etv[sl]
        pltpu.sync_copy(onesv, tbl_sp.at[segv], add=True)
        return 0

    lax.fori_loop(0, CNT_CHUNKS, count_body, 0)
    plsc.subcore_barrier()

    comp_lo = compv[pl.ds(0, LANES)]
    comp_hi = compv[pl.ds(LANES, LANES)]

    def tbl_body(j, _):
        lo = pl.ds(j * REL_PAD, LANES)
        hi = pl.ds(j * REL_PAD + LANES, LANES)
        tbuf[lo] = comp_lo / jnp.maximum(tbuf[lo], 1.0)
        tbuf[hi] = comp_hi / jnp.maximum(tbuf[hi], 1.0)
        return 0

    for q in range(TBL_PER_TILE // TBUF):
        tb = s * TBL_PER_TILE + q * TBUF
        pltpu.sync_copy(tbl_sp.at[pl.ds(tb, TBUF)], tbuf)
        lax.fori_loop(0, TBUF // REL_PAD, tbl_body, 0)
        pltpu.sync_copy(tbuf, tbl_sp.at[pl.ds(tb, TBUF)])
    plsc.subcore_barrier()

    ebase = c * (N_EDGES // NC) + s * EDGES_PER_TILE

    def main_body(i, _):
        base = ebase + i * CHUNK
        pltpu.sync_copy(src_hbm.at[pl.ds(base, CHUNK)], srcv)
        pltpu.sync_copy(dst_hbm.at[pl.ds(base, CHUNK)], dstv)
        pltpu.sync_copy(et_hbm.at[pl.ds(base, CHUNK)], etv)
        for j in range(CHUNK // LANES):
            sl = pl.ds(j * LANES, LANES)
            segv[sl] = lax.shift_left(dstv[sl], 5) + etv[sl]
        pltpu.sync_copy(tbl_sp.at[segv], coefv)
        pltpu.sync_copy(x_hbm.at[srcv], rows_v)

        def scale_body(e, _):
            cv = plsc.load_gather(coefv, [jnp.full((LANES,), e, jnp.int32)])
            for j in range(IN_CH // LANES):
                sl = pl.ds(j * LANES, LANES)
                rows_v[e, sl] = rows_v[e, sl] * cv
            return 0

        lax.fori_loop(0, CHUNK, scale_body, 0)
        pltpu.sync_copy(rows_v, z_sp.at[dstv], add=True)
        return 0

    lax.fori_loop(0, MAIN_CHUNKS, main_body, 0)
    plsc.subcore_barrier()

    for k in range(ROWS_PER_TILE // CHUNK):
        rb = s * ROWS_PER_TILE + k * CHUNK
        pltpu.sync_copy(z_sp.at[pl.ds(rb, CHUNK)], rows_v)
        pltpu.sync_copy(rows_v, out_hbm.at[c, pl.ds(rb, CHUNK)])


_sc_scatter = pl.kernel(
    _sc_body,
    out_type=jax.ShapeDtypeStruct((NC, N_PAD, IN_CH), jnp.float32),
    mesh=plsc.VectorSubcoreMesh(core_axis_name="c", subcore_axis_name="s",
                                num_cores=NC, num_subcores=NS),
    compiler_params=pltpu.CompilerParams(needs_layout_passes=False),
    scratch_types=[
        pltpu.VMEM_SHARED((N_PAD, IN_CH), jnp.float32),
        pltpu.VMEM_SHARED((SEGS,), jnp.float32),
        pltpu.VMEM((CHUNK, IN_CH), jnp.float32),
        pltpu.VMEM((CHUNK,), jnp.int32),
        pltpu.VMEM((CHUNK,), jnp.int32),
        pltpu.VMEM((CHUNK,), jnp.int32),
        pltpu.VMEM((CHUNK,), jnp.int32),
        pltpu.VMEM((CHUNK,), jnp.float32),
        pltpu.VMEM((CHUNK,), jnp.float32),
        pltpu.VMEM((REL_PAD,), jnp.float32),
        pltpu.VMEM((TBUF,), jnp.float32),
    ],
)


ROW_BLK = 1000
N_BLKS = N_NODES // ROW_BLK


def _mm_body(z_ref, x_ref, b_ref, r_ref, bias_ref, out_ref, stats_ref,
             acc_ref):
    i = pl.program_id(0)
    zr = z_ref[0] + z_ref[1]
    op = (jnp.dot(zr, b_ref[...], preferred_element_type=jnp.float32)
          + jnp.dot(x_ref[...], r_ref[...], preferred_element_type=jnp.float32)
          + bias_ref[...])
    out_ref[...] = op
    ssum = jnp.sum(op, axis=0, keepdims=True)
    ssq = jnp.sum(op * op, axis=0, keepdims=True)
    st = jnp.concatenate([ssum, ssq], axis=0)

    @pl.when(i == 0)
    def _():
        acc_ref[...] = st

    @pl.when(i > 0)
    def _():
        acc_ref[...] = acc_ref[...] + st

    @pl.when(i == N_BLKS - 1)
    def _():
        stats_ref[...] = acc_ref[...]


def _mm_call(z2, x, basis0, root, bias2d):
    return pl.pallas_call(
        _mm_body,
        grid=(N_BLKS,),
        in_specs=[
            pl.BlockSpec((NC, ROW_BLK, IN_CH), lambda i: (0, i, 0)),
            pl.BlockSpec((ROW_BLK, IN_CH), lambda i: (i, 0)),
            pl.BlockSpec((IN_CH, HID_CH), lambda i: (0, 0)),
            pl.BlockSpec((IN_CH, HID_CH), lambda i: (0, 0)),
            pl.BlockSpec((1, HID_CH), lambda i: (0, 0)),
        ],
        out_specs=[
            pl.BlockSpec((ROW_BLK, HID_CH), lambda i: (i, 0)),
            pl.BlockSpec((2, HID_CH), lambda i: (0, 0)),
        ],
        out_shape=[
            jax.ShapeDtypeStruct((N_NODES, HID_CH), jnp.float32),
            jax.ShapeDtypeStruct((2, HID_CH), jnp.float32),
        ],
        scratch_shapes=[pltpu.VMEM((2, HID_CH), jnp.float32)],
    )(z2, x, basis0, root, bias2d)


def _bn_body(op_ref, stats_ref, g_ref, b_ref, out_ref):
    n = jnp.float32(N_NODES)
    mean = stats_ref[0:1, :] / n
    var = stats_ref[1:2, :] / n - mean * mean
    inv = lax.rsqrt(var + 1e-5)
    y = (op_ref[...] - mean) * (inv * g_ref[...]) + b_ref[...]
    out_ref[...] = jnp.maximum(y, 0.0)


def _bn_call(op, stats, gamma2d, beta2d):
    return pl.pallas_call(
        _bn_body,
        grid=(N_BLKS,),
        in_specs=[
            pl.BlockSpec((ROW_BLK, HID_CH), lambda i: (i, 0)),
            pl.BlockSpec((2, HID_CH), lambda i: (0, 0)),
            pl.BlockSpec((1, HID_CH), lambda i: (0, 0)),
            pl.BlockSpec((1, HID_CH), lambda i: (0, 0)),
        ],
        out_specs=pl.BlockSpec((ROW_BLK, HID_CH), lambda i: (i, 0)),
        out_shape=jax.ShapeDtypeStruct((N_NODES, HID_CH), jnp.float32),
    )(op, stats, gamma2d, beta2d)


def kernel(x, edge_index, edge_type, basis, comp, root, bias, gamma, beta):
    src = edge_index[0].astype(jnp.int32)
    dst = edge_index[1].astype(jnp.int32)
    et = edge_type.astype(jnp.int32)
    comp_pad = jnp.zeros((REL_PAD,), jnp.float32).at[:NUM_REL].set(comp[:, 0])
    z2 = _sc_scatter(src, dst, et, comp_pad, x)
    op, stats = _mm_call(z2, x, basis[0], root,
                         bias.reshape(1, HID_CH))
    return _bn_call(op, stats, gamma.reshape(1, HID_CH),
                    beta.reshape(1, HID_CH))


# speedup vs baseline: 23.9458x; 2.8910x over previous
import functools

import jax
import jax.numpy as jnp
from jax import lax
from jax.experimental import pallas as pl
from jax.experimental.pallas import tpu as pltpu
from jax.experimental.pallas import tpu_sc as plsc

N_NODES = 10000
N_EDGES = 320000
IN_CH = 128
HID_CH = 256
NUM_REL = 24
REL_PAD = 32
SEGS = N_NODES * REL_PAD
N_PAD = 10240

NC = 2
NS = 16
LANES = 16
CHUNK = 80
SPLIT_A = 48
SPLIT_B = 32
CHUNK1 = 128
CNT_FULL = (N_EDGES // NS) // CHUNK1
CNT_TAIL = (N_EDGES // NS) - CNT_FULL * CHUNK1

EDGES_PER_TILE_CNT = N_EDGES // NS
EDGES_PER_TILE = N_EDGES // (NC * NS)
CNT_CHUNKS = EDGES_PER_TILE_CNT // CHUNK
MAIN_CHUNKS = EDGES_PER_TILE // CHUNK
TBL_PER_TILE = SEGS // NS
TBUF = 4000
ROWS_PER_TILE = N_PAD // NS


def _sc_body(src_hbm, dst_hbm, et_hbm, comp_hbm, x_hbm,
             out_hbm, z_sp, tbl_sp, rows_v, srcv, dstv, etv, segv, segr,
             segt, coefv, sdstA, sdstB, onesv, compv, tbuf, ld_sem, g_sem,
             cf_sem, sc_sem, cs_sem):
    c = lax.axis_index("c")
    s = lax.axis_index("s")

    zero = jnp.zeros((LANES,), jnp.float32)
    r0 = rows_v.at[0]

    def zero_rows(e, _):
        for j in range(IN_CH // LANES):
            r0[e, pl.ds(j * LANES, LANES)] = zero
        return 0

    lax.fori_loop(0, CHUNK, zero_rows, 0)

    def zero_tbuf(j, _):
        tbuf[pl.ds(j * LANES, LANES)] = zero
        return 0

    lax.fori_loop(0, TBUF // LANES, zero_tbuf, 0)
    for q in range(TBL_PER_TILE // TBUF):
        pltpu.async_copy(tbuf, tbl_sp.at[pl.ds(s * TBL_PER_TILE + q * TBUF,
                                               TBUF)], g_sem.at[0])
    for k in range(ROWS_PER_TILE // CHUNK):
        pltpu.async_copy(
            r0, z_sp.at[pl.ds(s * ROWS_PER_TILE + k * CHUNK, CHUNK)],
            g_sem.at[1])
    pltpu.sync_copy(comp_hbm, compv)
    for q in range(TBL_PER_TILE // TBUF):
        pltpu.make_async_copy(
            tbuf, tbl_sp.at[pl.ds(s * TBL_PER_TILE + q * TBUF, TBUF)],
            g_sem.at[0]).wait()
    for k in range(ROWS_PER_TILE // CHUNK):
        pltpu.make_async_copy(
            r0, z_sp.at[pl.ds(s * ROWS_PER_TILE + k * CHUNK, CHUNK)],
            g_sem.at[1]).wait()
    ones = jnp.ones((LANES,), jnp.float32)
    for j in range(CHUNK1 // LANES):
        onesv[pl.ds(j * LANES, LANES)] = ones
    plsc.subcore_barrier()

    def cnt_base(ci):
        return s * EDGES_PER_TILE_CNT + ci * CHUNK1

    def cnt_issue(ci, p):
        base = cnt_base(ci)
        pltpu.async_copy(dst_hbm.at[pl.ds(base, CHUNK1)], dstv.at[p],
                         ld_sem.at[p])
        pltpu.async_copy(et_hbm.at[pl.ds(base, CHUNK1)], etv.at[p],
                         ld_sem.at[p])

    def cnt_wait_scatter(r):
        pltpu.make_async_copy(onesv, tbl_sp.at[segr.at[r]],
                              cs_sem.at[r]).wait()

    def cnt_proc(ci, p, r, first):
        base = cnt_base(ci)
        pltpu.make_async_copy(dst_hbm.at[pl.ds(base, CHUNK1)], dstv.at[p],
                              ld_sem.at[p]).wait()
        pltpu.make_async_copy(et_hbm.at[pl.ds(base, CHUNK1)], etv.at[p],
                              ld_sem.at[p]).wait()
        if not first:
            cnt_wait_scatter(r)
        for j in range(CHUNK1 // LANES):
            sl = pl.ds(j * LANES, LANES)
            segr[r, sl] = lax.shift_left(dstv[p, sl], 5) + etv[p, sl]
        pltpu.async_copy(onesv, tbl_sp.at[segr.at[r]], cs_sem.at[r],
                         add=True)

    cnt_issue(0, 0)
    cnt_issue(1, 1)
    for k in range(4):
        cnt_proc(k, k % 2, k, True)
        cnt_issue(k + 2, k % 2)

    def cnt_loop(i, _):
        for h in range(4):
            ci = 4 + 4 * i + h
            cnt_proc(ci, ci % 2, h, False)

            @pl.when(ci + 2 <= CNT_FULL - 1)
            def _():
                cnt_issue(ci + 2, ci % 2)

        return 0

    lax.fori_loop(0, (CNT_FULL - 4) // 4, cnt_loop, 0)
    tbase = cnt_base(CNT_FULL)
    pltpu.sync_copy(dst_hbm.at[pl.ds(tbase, CNT_TAIL)],
                    dstv.at[0, pl.ds(0, CNT_TAIL)])
    pltpu.sync_copy(et_hbm.at[pl.ds(tbase, CNT_TAIL)],
                    etv.at[0, pl.ds(0, CNT_TAIL)])
    for r in range(4):
        cnt_wait_scatter(r)
    for j in range(CNT_TAIL // LANES):
        sl = pl.ds(j * LANES, LANES)
        segt[sl] = lax.shift_left(dstv[0, sl], 5) + etv[0, sl]
    pltpu.sync_copy(onesv.at[pl.ds(0, CNT_TAIL)], tbl_sp.at[segt], add=True)
    plsc.subcore_barrier()

    comp_lo = compv[pl.ds(0, LANES)]
    comp_hi = compv[pl.ds(LANES, LANES)]

    def tbl_body(j, _):
        lo = pl.ds(j * REL_PAD, LANES)
        hi = pl.ds(j * REL_PAD + LANES, LANES)
        tbuf[lo] = comp_lo / jnp.maximum(tbuf[lo], 1.0)
        tbuf[hi] = comp_hi / jnp.maximum(tbuf[hi], 1.0)
        return 0

    for q in range(TBL_PER_TILE // TBUF):
        tb = s * TBL_PER_TILE + q * TBUF
        pltpu.sync_copy(tbl_sp.at[pl.ds(tb, TBUF)], tbuf)
        lax.fori_loop(0, TBUF // REL_PAD, tbl_body, 0)
        pltpu.sync_copy(tbuf, tbl_sp.at[pl.ds(tb, TBUF)])
    plsc.subcore_barrier()

    ebase = c * (N_EDGES // NC) + s * EDGES_PER_TILE

    def m_base(ci):
        return ebase + ci * CHUNK

    def m_issue(ci, p):
        base = m_base(ci)
        pltpu.async_copy(src_hbm.at[pl.ds(base, CHUNK)], srcv.at[p],
                         ld_sem.at[p])
        pltpu.async_copy(dst_hbm.at[pl.ds(base, CHUNK)],
                         dstv.at[p, pl.ds(0, CHUNK)], ld_sem.at[p])
        pltpu.async_copy(et_hbm.at[pl.ds(base, CHUNK)],
                         etv.at[p, pl.ds(0, CHUNK)], ld_sem.at[p])

    def m_wait_scatter(p):
        pltpu.make_async_copy(rows_v.at[p, pl.ds(0, SPLIT_A)],
                              z_sp.at[sdstA.at[p]], sc_sem.at[p]).wait()
        pltpu.make_async_copy(rows_v.at[p, pl.ds(SPLIT_A, SPLIT_B)],
                              z_sp.at[sdstB.at[p]], sc_sem.at[p]).wait()

    def m_gather(ci, p, first):
        base = m_base(ci)
        pltpu.make_async_copy(src_hbm.at[pl.ds(base, CHUNK)], srcv.at[p],
                              ld_sem.at[p]).wait()
        pltpu.make_async_copy(dst_hbm.at[pl.ds(base, CHUNK)],
                              dstv.at[p, pl.ds(0, CHUNK)],
                              ld_sem.at[p]).wait()
        pltpu.make_async_copy(et_hbm.at[pl.ds(base, CHUNK)],
                              etv.at[p, pl.ds(0, CHUNK)],
                              ld_sem.at[p]).wait()
        if not first:
            m_wait_scatter(p)
        for j in range(CHUNK // LANES):
            sl = pl.ds(j * LANES, LANES)
            segv[p, sl] = lax.shift_left(dstv[p, sl], 5) + etv[p, sl]
            if j < SPLIT_A // LANES:
                sdstA[p, pl.ds(j * LANES, LANES)] = dstv[p, sl]
            else:
                sdstB[p, pl.ds((j - SPLIT_A // LANES) * LANES, LANES)] = (
                    dstv[p, sl])
        pltpu.async_copy(tbl_sp.at[segv.at[p]], coefv.at[p], cf_sem.at[p])
        pltpu.async_copy(x_hbm.at[srcv.at[p]], rows_v.at[p], g_sem.at[p])

    def m_proc(ci, p):
        pltpu.make_async_copy(tbl_sp.at[segv.at[p]], coefv.at[p],
                              cf_sem.at[p]).wait()
        pltpu.make_async_copy(x_hbm.at[srcv.at[p]], rows_v.at[p],
                              g_sem.at[p]).wait()
        rv = rows_v.at[p]
        cf = coefv.at[p]

        def scale2(base):
            def body(i, _):
                for d in range(2):
                    e = base + 2 * i + d
                    cv = plsc.load_gather(
                        cf, [jnp.full((LANES,), e, jnp.int32)])
                    for j in range(IN_CH // LANES):
                        sl = pl.ds(j * LANES, LANES)
                        rv[e, sl] = rv[e, sl] * cv
                return 0
            return body

        lax.fori_loop(0, SPLIT_A // 2, scale2(0), 0)
        pltpu.async_copy(rv.at[pl.ds(0, SPLIT_A)], z_sp.at[sdstA.at[p]],
                         sc_sem.at[p], add=True)
        lax.fori_loop(0, SPLIT_B // 2, scale2(SPLIT_A), 0)
        pltpu.async_copy(rv.at[pl.ds(SPLIT_A, SPLIT_B)], z_sp.at[sdstB.at[p]],
                         sc_sem.at[p], add=True)

    m_issue(0, 0)
    m_gather(0, 0, True)
    m_issue(1, 1)
    m_gather(1, 1, True)
    m_proc(0, 0)
    m_issue(2, 0)

    def m_loop(i, _):
        for h in range(2):
            ci = 2 * i + h
            p = h
            m_gather(ci, p, False)
            m_proc(ci - 1, 1 - p)

            @pl.when(ci + 1 <= MAIN_CHUNKS - 1)
            def _():
                m_issue(ci + 1, 1 - p)

        return 0

    def m_loop_shift(i, _):
        return m_loop(i + 1, _)

    lax.fori_loop(0, (MAIN_CHUNKS - 3) // 2, m_loop_shift, 0)
    m_gather(MAIN_CHUNKS - 1, 0, False)
    m_proc(MAIN_CHUNKS - 2, 1)
    m_proc(MAIN_CHUNKS - 1, 0)
    m_wait_scatter(0)
    m_wait_scatter(1)
    plsc.subcore_barrier()

    NK = ROWS_PER_TILE // CHUNK

    def p4_in(k, p):
        pltpu.async_copy(z_sp.at[pl.ds(s * ROWS_PER_TILE + k * CHUNK, CHUNK)],
                         rows_v.at[p], ld_sem.at[p])

    def p4_wait_in(k, p):
        pltpu.make_async_copy(
            z_sp.at[pl.ds(s * ROWS_PER_TILE + k * CHUNK, CHUNK)],
            rows_v.at[p], ld_sem.at[p]).wait()

    def p4_out(k, p):
        pltpu.async_copy(rows_v.at[p],
                         out_hbm.at[c, pl.ds(s * ROWS_PER_TILE + k * CHUNK,
                                             CHUNK)], g_sem.at[p])

    def p4_wait_out(k, p):
        pltpu.make_async_copy(
            rows_v.at[p],
            out_hbm.at[c, pl.ds(s * ROWS_PER_TILE + k * CHUNK, CHUNK)],
            g_sem.at[p]).wait()

    p4_in(0, 0)
    p4_in(1, 1)
    for k in range(NK):
        p = k % 2
        if k >= 2:
            p4_wait_out(k - 2, p)
            p4_in(k, p)
        p4_wait_in(k, p)
        p4_out(k, p)
    p4_wait_out(NK - 2, 0)
    p4_wait_out(NK - 1, 1)


_sc_scatter = pl.kernel(
    _sc_body,
    out_type=jax.ShapeDtypeStruct((NC, N_PAD, IN_CH), jnp.float32),
    mesh=plsc.VectorSubcoreMesh(core_axis_name="c", subcore_axis_name="s",
                                num_cores=NC, num_subcores=NS),
    compiler_params=pltpu.CompilerParams(needs_layout_passes=False),
    scratch_types=[
        pltpu.VMEM_SHARED((N_PAD, IN_CH), jnp.float32),
        pltpu.VMEM_SHARED((SEGS,), jnp.float32),
        pltpu.VMEM((2, CHUNK, IN_CH), jnp.float32),
        pltpu.VMEM((2, CHUNK), jnp.int32),
        pltpu.VMEM((2, CHUNK1), jnp.int32),
        pltpu.VMEM((2, CHUNK1), jnp.int32),
        pltpu.VMEM((2, CHUNK), jnp.int32),
        pltpu.VMEM((4, CHUNK1), jnp.int32),
        pltpu.VMEM((CNT_TAIL,), jnp.int32),
        pltpu.VMEM((2, CHUNK), jnp.float32),
        pltpu.VMEM((2, SPLIT_A), jnp.int32),
        pltpu.VMEM((2, SPLIT_B), jnp.int32),
        pltpu.VMEM((CHUNK1,), jnp.float32),
        pltpu.VMEM((REL_PAD,), jnp.float32),
        pltpu.VMEM((TBUF,), jnp.float32),
        pltpu.SemaphoreType.DMA((2,)),
        pltpu.SemaphoreType.DMA((2,)),
        pltpu.SemaphoreType.DMA((2,)),
        pltpu.SemaphoreType.DMA((2,)),
        pltpu.SemaphoreType.DMA((4,)),
    ],
)


def _xr_body(x_ref, r_ref, bias_ref, out_ref):
    out_ref[...] = (jnp.dot(x_ref[...], r_ref[...],
                            preferred_element_type=jnp.float32)
                    + bias_ref[...])


def _xr_call(x, root, bias2d):
    return pl.pallas_call(
        _xr_body,
        out_shape=jax.ShapeDtypeStruct((N_NODES, HID_CH), jnp.float32),
    )(x, root, bias2d)


def _fused_body(z_ref, xr_ref, b_ref, g_ref, be_ref, out_ref):
    zz = z_ref[0, :N_NODES, :] + z_ref[1, :N_NODES, :]
    op = (jnp.dot(zz, b_ref[...], preferred_element_type=jnp.float32)
          + xr_ref[...])
    n = jnp.float32(N_NODES)
    mean = jnp.sum(op, axis=0, keepdims=True) / n
    var = jnp.sum(op * op, axis=0, keepdims=True) / n - mean * mean
    inv = lax.rsqrt(var + 1e-5)
    out_ref[...] = jnp.maximum((op - mean) * (inv * g_ref[...]) + be_ref[...],
                               0.0)


def _fused_call(z2, xr, basis0, gamma2d, beta2d):
    return pl.pallas_call(
        _fused_body,
        out_shape=jax.ShapeDtypeStruct((N_NODES, HID_CH), jnp.float32),
    )(z2, xr, basis0, gamma2d, beta2d)


def kernel(x, edge_index, edge_type, basis, comp, root, bias, gamma, beta):
    src = edge_index[0].astype(jnp.int32)
    dst = edge_index[1].astype(jnp.int32)
    et = edge_type.astype(jnp.int32)
    comp_pad = jnp.zeros((REL_PAD,), jnp.float32).at[:NUM_REL].set(comp[:, 0])
    z2 = _sc_scatter(src, dst, et, comp_pad, x)
    xr = _xr_call(x, root, bias.reshape(1, HID_CH))
    return _fused_call(z2, xr, basis[0], gamma.reshape(1, HID_CH),
                       beta.reshape(1, HID_CH))
